# Initial kernel scaffold; baseline (speedup 1.0000x reference)
import jax
import jax.numpy as jnp
from jax.experimental import pallas as pl


def kernel(x, edge_index, ep_w1, ep_b1, ep_w2, ep_b2, cheb_w, cheb_b):
    raise NotImplementedError("write your pallas kernel here")


# speedup vs baseline: 1.1882x; 1.1882x over previous
import functools

import jax
import jax.numpy as jnp
from jax.experimental import pallas as pl

N = 10000
E = 320000
D = 128
H = 128

ROW_BLK = 1000
EDGE_BLK = 16000


def _pq_body(x_ref, w1a_ref, w1b_ref, b1_ref, p_ref, q_ref):
    x = x_ref[...]
    p_ref[...] = jnp.dot(x, w1a_ref[...].T, preferred_element_type=jnp.float32)
    q_ref[...] = (
        jnp.dot(x, w1b_ref[...].T, preferred_element_type=jnp.float32)
        + b1_ref[...]
    )


def _compute_pq(x, ep_w1, ep_b1):
    w1a = ep_w1[:, :D]
    w1b = ep_w1[:, D:]
    grid = (N // ROW_BLK,)
    return pl.pallas_call(
        _pq_body,
        grid=grid,
        in_specs=[
            pl.BlockSpec((ROW_BLK, D), lambda i: (i, 0)),
            pl.BlockSpec((H, D), lambda i: (0, 0)),
            pl.BlockSpec((H, D), lambda i: (0, 0)),
            pl.BlockSpec((H,), lambda i: (0,)),
        ],
        out_specs=[
            pl.BlockSpec((ROW_BLK, H), lambda i: (i, 0)),
            pl.BlockSpec((ROW_BLK, H), lambda i: (i, 0)),
        ],
        out_shape=[
            jax.ShapeDtypeStruct((N, H), jnp.float32),
            jax.ShapeDtypeStruct((N, H), jnp.float32),
        ],
    )(x, w1a, w1b, ep_b1)


def _edge_body(pr_ref, qc_ref, w2_ref, b2_ref, w_ref):
    h = jnp.maximum(pr_ref[...] + qc_ref[...], 0.0)
    z = jnp.sum(h * w2_ref[...], axis=1) + b2_ref[0]
    w = jax.nn.sigmoid(z)
    w_ref[...] = w.reshape(w_ref.shape)


def _edge_weights(pr, qc, ep_w2, ep_b2):
    nblk = E // EDGE_BLK
    out = pl.pallas_call(
        _edge_body,
        grid=(nblk,),
        in_specs=[
            pl.BlockSpec((EDGE_BLK, H), lambda i: (i, 0)),
            pl.BlockSpec((EDGE_BLK, H), lambda i: (i, 0)),
            pl.BlockSpec((1, H), lambda i: (0, 0)),
            pl.BlockSpec((1,), lambda i: (0,)),
        ],
        out_specs=pl.BlockSpec((EDGE_BLK // 2000, 2000), lambda i: (i, 0)),
        out_shape=jax.ShapeDtypeStruct((E // 2000, 2000), jnp.float32),
    )(pr, qc, ep_w2, ep_b2)
    return out.reshape(E)


def _cheb_body(x_ref, t1_ref, t2_ref, w0_ref, w1_ref, w2_ref, b_ref, o_ref):
    acc = jnp.dot(x_ref[...], w0_ref[...].T, preferred_element_type=jnp.float32)
    acc += jnp.dot(t1_ref[...], w1_ref[...].T, preferred_element_type=jnp.float32)
    acc += jnp.dot(t2_ref[...], w2_ref[...].T, preferred_element_type=jnp.float32)
    o_ref[...] = acc + b_ref[...]


def _cheb_out(x, tx1, tx2, cheb_w, cheb_b):
    grid = (N // ROW_BLK,)
    wspec = pl.BlockSpec((D, D), lambda i: (0, 0))
    return pl.pallas_call(
        _cheb_body,
        grid=grid,
        in_specs=[
            pl.BlockSpec((ROW_BLK, D), lambda i: (i, 0)),
            pl.BlockSpec((ROW_BLK, D), lambda i: (i, 0)),
            pl.BlockSpec((ROW_BLK, D), lambda i: (i, 0)),
            wspec,
            wspec,
            wspec,
            pl.BlockSpec((D,), lambda i: (0,)),
        ],
        out_specs=pl.BlockSpec((ROW_BLK, D), lambda i: (i, 0)),
        out_shape=jax.ShapeDtypeStruct((N, D), jnp.float32),
    )(x, tx1, tx2, cheb_w[0], cheb_w[1], cheb_w[2], cheb_b)


@jax.jit
def kernel(x, edge_index, ep_w1, ep_b1, ep_w2, ep_b2, cheb_w, cheb_b):
    row, col = edge_index[0], edge_index[1]

    p, q = _compute_pq(x, ep_w1, ep_b1)
    edge_weights = _edge_weights(p[row], q[col], ep_w2, ep_b2)

    deg = jnp.zeros((N,), jnp.float32).at[row].add(edge_weights)
    dis = jnp.where(deg > 0, jax.lax.rsqrt(deg), 0.0)
    val = -dis[row] * edge_weights * dis[col]
    val = jnp.where(row == col, val - 1.0, val)

    def spmv(hh):
        msg = val[:, None] * hh[row]
        return jnp.zeros_like(hh).at[col].add(msg)

    tx1 = spmv(x)
    tx2 = 2.0 * spmv(tx1) - x
    out = _cheb_out(x, tx1, tx2, cheb_w, cheb_b)
    return (out, edge_weights)


# speedup vs baseline: 4.3867x; 3.6918x over previous
import functools

import jax
import jax.numpy as jnp
from jax import lax
from jax.experimental import pallas as pl
from jax.experimental.pallas import tpu as pltpu
from jax.experimental.pallas import tpu_sc as plsc

N = 10000
E = 320000
D = 128
H = 128

NC = 2
NS = 16
NW = NC * NS
NP = 10240
ROWS_PT = NP // NS
EW = E // NW
K = 80
NCHUNK = EW // K

ROW_BLK = 1000

_mesh = plsc.VectorSubcoreMesh(core_axis_name="c", subcore_axis_name="s")
_f32 = jnp.float32


def _pq_body(x_ref, w1a_ref, w1b_ref, b1_ref, p_ref, q_ref):
    x = x_ref[...]
    p_ref[...] = jnp.dot(x, w1a_ref[...].T, preferred_element_type=_f32)
    q_ref[...] = (
        jnp.dot(x, w1b_ref[...].T, preferred_element_type=_f32) + b1_ref[...]
    )


def _compute_pq(x, ep_w1, ep_b1):
    return pl.pallas_call(
        _pq_body,
        grid=(N // ROW_BLK,),
        in_specs=[
            pl.BlockSpec((ROW_BLK, D), lambda i: (i, 0)),
            pl.BlockSpec((H, D), lambda i: (0, 0)),
            pl.BlockSpec((H, D), lambda i: (0, 0)),
            pl.BlockSpec((H,), lambda i: (0,)),
        ],
        out_specs=[
            pl.BlockSpec((ROW_BLK, H), lambda i: (i, 0)),
            pl.BlockSpec((ROW_BLK, H), lambda i: (i, 0)),
        ],
        out_shape=[
            jax.ShapeDtypeStruct((N, H), _f32),
            jax.ShapeDtypeStruct((N, H), _f32),
        ],
    )(x, ep_w1[:, :D], ep_w1[:, D:], ep_b1)


def _mid_body(x_ref, p0_ref, p1_ref, w0_ref, w1_ref, tx1_ref, o_ref):
    t1 = p0_ref[...] + p1_ref[...]
    tx1_ref[...] = t1
    o_ref[...] = (
        jnp.dot(x_ref[...], w0_ref[...].T, preferred_element_type=_f32)
        + jnp.dot(t1, w1_ref[...].T, preferred_element_type=_f32)
    )


def _tc_mid(x, p0, p1, w0, w1):
    wspec = pl.BlockSpec((D, D), lambda i: (0, 0))
    rspec = pl.BlockSpec((ROW_BLK, D), lambda i: (i, 0))
    return pl.pallas_call(
        _mid_body,
        grid=(N // ROW_BLK,),
        in_specs=[rspec, rspec, rspec, wspec, wspec],
        out_specs=[rspec, rspec],
        out_shape=[
            jax.ShapeDtypeStruct((N, D), _f32),
            jax.ShapeDtypeStruct((N, D), _f32),
        ],
    )(x, p0, p1, w0, w1)


def _final_body(x_ref, o01_ref, q0_ref, q1_ref, w2_ref, b_ref, o_ref):
    t2 = 2.0 * (q0_ref[...] + q1_ref[...]) - x_ref[...]
    o_ref[...] = (
        o01_ref[...]
        + jnp.dot(t2, w2_ref[...].T, preferred_element_type=_f32)
        + b_ref[...]
    )


def _tc_final(x, o01, q0, q1, w2, b):
    wspec = pl.BlockSpec((D, D), lambda i: (0, 0))
    rspec = pl.BlockSpec((ROW_BLK, D), lambda i: (i, 0))
    return pl.pallas_call(
        _final_body,
        grid=(N // ROW_BLK,),
        in_specs=[rspec, rspec, rspec, rspec, wspec,
                  pl.BlockSpec((D,), lambda i: (0,))],
        out_specs=rspec,
        out_shape=jax.ShapeDtypeStruct((N, D), _f32),
    )(x, o01, q0, q1, w2, b)


def _rsqrt16(d):
    i = plsc.bitcast(d, jnp.int32)
    i = 0x5F3759DF - lax.shift_right_arithmetic(i, 1)
    y = plsc.bitcast(i, _f32)
    for _ in range(3):
        y = y * (1.5 - 0.5 * d * y * y)
    return y


def _sc_edge_body(p_hbm, q_hbm, row_hbm, col_hbm, w2_hbm, b2_hbm, zn_hbm,
                  w_hbm, degp_hbm,
                  idxr, idxc, pbuf, qbuf, wbuf, w2v, b2v, deg_sh,
                  sem1, sem2):
    c = lax.axis_index("c")
    s = lax.axis_index("s")
    wid = s * NC + c
    pltpu.sync_copy(w2_hbm, w2v)
    pltpu.sync_copy(b2_hbm, b2v)
    sl_tile = pl.ds(s * ROWS_PT, ROWS_PT)
    pltpu.sync_copy(zn_hbm.at[sl_tile], deg_sh.at[sl_tile])
    plsc.subcore_barrier()

    def chunk(i, carry):
        base = wid * EW + i * K
        pltpu.sync_copy(row_hbm.at[pl.ds(base, K)], idxr)
        pltpu.sync_copy(col_hbm.at[pl.ds(base, K)], idxc)
        cp1 = pltpu.async_copy(p_hbm.at[idxr], pbuf, sem1)
        cp2 = pltpu.async_copy(q_hbm.at[idxc], qbuf, sem2)
        cp1.wait()
        cp2.wait()

        def group(g, carry2):
            gbase = g * 16
            e16 = gbase + jnp.arange(16, dtype=jnp.int32)
            acc = b2v[...]
            for j in range(8):
                w2j = w2v[pl.ds(j * 16, 16)]
                for i in range(16):
                    fi = jnp.full((16,), j * 16 + i, jnp.int32)
                    pv = plsc.load_gather(pbuf, [e16, fi])
                    qv = plsc.load_gather(qbuf, [e16, fi])
                    acc = acc + jnp.maximum(pv + qv, 0.0) * w2j[i]
            wbuf[pl.ds(gbase, 16)] = 1.0 / (1.0 + jnp.exp(-acc))
            return carry2

        lax.fori_loop(0, K // 16, group, 0)
        pltpu.sync_copy(wbuf, w_hbm.at[pl.ds(base, K)])
        pltpu.sync_copy(wbuf, deg_sh.at[idxr], add=True)
        return carry

    lax.fori_loop(0, NCHUNK, chunk, 0)
    plsc.subcore_barrier()
    pltpu.sync_copy(deg_sh.at[sl_tile], degp_hbm.at[c, sl_tile])


def _sc_edge(p, q, row, col, w2, b2x16, zn):
    return pl.kernel(
        _sc_edge_body,
        out_type=[
            jax.ShapeDtypeStruct((E,), _f32),
            jax.ShapeDtypeStruct((NC, NP), _f32),
        ],
        mesh=_mesh,
        compiler_params=pltpu.CompilerParams(needs_layout_passes=False),
        scratch_types=[
            pltpu.VMEM((K,), jnp.int32),
            pltpu.VMEM((K,), jnp.int32),
            pltpu.VMEM((K, H), _f32),
            pltpu.VMEM((K, H), _f32),
            pltpu.VMEM((K,), _f32),
            pltpu.VMEM((H,), _f32),
            pltpu.VMEM((16,), _f32),
            pltpu.VMEM_SHARED((NP,), _f32),
            pltpu.SemaphoreType.DMA,
            pltpu.SemaphoreType.DMA,
        ],
    )(p, q, row, col, w2, b2x16, zn)


def _sc_spmv1_body(degp_hbm, w_hbm, row_hbm, col_hbm, x_hbm, znd_hbm,
                   val_hbm, txp_hbm,
                   disv, degb, idxr, idxc, wv, valv, xbuf, tx_sh, sem1):
    c = lax.axis_index("c")
    s = lax.axis_index("s")
    wid = s * NC + c
    sl_tile = pl.ds(s * ROWS_PT, ROWS_PT)
    pltpu.sync_copy(znd_hbm.at[sl_tile], tx_sh.at[sl_tile])
    pltpu.sync_copy(degp_hbm.at[0], disv)
    pltpu.sync_copy(degp_hbm.at[1], degb)

    def dis_step(t, carry):
        sl = pl.ds(t * 16, 16)
        d = disv[sl] + degb[sl]
        disv[sl] = jnp.where(d > 0.0, _rsqrt16(d), 0.0)
        return carry

    lax.fori_loop(0, NP // 16, dis_step, 0)
    plsc.subcore_barrier()

    def chunk(i, carry):
        base = wid * EW + i * K
        pltpu.sync_copy(row_hbm.at[pl.ds(base, K)], idxr)
        pltpu.sync_copy(col_hbm.at[pl.ds(base, K)], idxc)
        pltpu.sync_copy(w_hbm.at[pl.ds(base, K)], wv)
        pltpu.async_copy(x_hbm.at[idxr], xbuf, sem1).wait()
        for g in range(K // 16):
            sl = pl.ds(g * 16, 16)
            r16 = idxr[sl]
            c16 = idxc[sl]
            dr = plsc.load_gather(disv, [r16])
            dc = plsc.load_gather(disv, [c16])
            v16 = -(dr * wv[sl] * dc)
            valv[sl] = jnp.where(r16 == c16, v16 - 1.0, v16)
        pltpu.sync_copy(valv, val_hbm.at[pl.ds(base, K)])

        @plsc.parallel_loop(0, K // 16, 1, unroll=1)
        def scale(g):
            v16 = valv[pl.ds(g * 16, 16)]
            for l in range(16):
                sv = v16[l]
                e = g * 16 + l
                for j in range(8):
                    sl = pl.ds(j * 16, 16)
                    xbuf[e, sl] = xbuf[e, sl] * sv

        pltpu.sync_copy(xbuf, tx_sh.at[idxc], add=True)
        return carry

    lax.fori_loop(0, NCHUNK, chunk, 0)
    plsc.subcore_barrier()
    pltpu.sync_copy(tx_sh.at[sl_tile], txp_hbm.at[c, sl_tile])


def _sc_spmv1(degp, w, row, col, x, znd):
    return pl.kernel(
        _sc_spmv1_body,
        out_type=[
            jax.ShapeDtypeStruct((E,), _f32),
            jax.ShapeDtypeStruct((NC, NP, D), _f32),
        ],
        mesh=_mesh,
        compiler_params=pltpu.CompilerParams(needs_layout_passes=False),
        scratch_types=[
            pltpu.VMEM((NP,), _f32),
            pltpu.VMEM((NP,), _f32),
            pltpu.VMEM((K,), jnp.int32),
            pltpu.VMEM((K,), jnp.int32),
            pltpu.VMEM((K,), _f32),
            pltpu.VMEM((K,), _f32),
            pltpu.VMEM((K, D), _f32),
            pltpu.VMEM_SHARED((NP, D), _f32),
            pltpu.SemaphoreType.DMA,
        ],
    )(degp, w, row, col, x, znd)


def _sc_spmv2_body(val_hbm, row_hbm, col_hbm, h_hbm, znd_hbm,
                   txp_hbm,
                   idxr, idxc, valv, xbuf, tx_sh, sem1):
    c = lax.axis_index("c")
    s = lax.axis_index("s")
    wid = s * NC + c
    sl_tile = pl.ds(s * ROWS_PT, ROWS_PT)
    pltpu.sync_copy(znd_hbm.at[sl_tile], tx_sh.at[sl_tile])
    plsc.subcore_barrier()

    def chunk(i, carry):
        base = wid * EW + i * K
        pltpu.sync_copy(row_hbm.at[pl.ds(base, K)], idxr)
        pltpu.sync_copy(col_hbm.at[pl.ds(base, K)], idxc)
        pltpu.sync_copy(val_hbm.at[pl.ds(base, K)], valv)
        pltpu.async_copy(h_hbm.at[idxr], xbuf, sem1).wait()

        @plsc.parallel_loop(0, K // 16, 1, unroll=1)
        def scale(g):
            v16 = valv[pl.ds(g * 16, 16)]
            for l in range(16):
                sv = v16[l]
                e = g * 16 + l
                for j in range(8):
                    sl = pl.ds(j * 16, 16)
                    xbuf[e, sl] = xbuf[e, sl] * sv

        pltpu.sync_copy(xbuf, tx_sh.at[idxc], add=True)
        return carry

    lax.fori_loop(0, NCHUNK, chunk, 0)
    plsc.subcore_barrier()
    pltpu.sync_copy(tx_sh.at[sl_tile], txp_hbm.at[c, sl_tile])


def _sc_spmv2(val, row, col, h, znd):
    return pl.kernel(
        _sc_spmv2_body,
        out_type=jax.ShapeDtypeStruct((NC, NP, D), _f32),
        mesh=_mesh,
        compiler_params=pltpu.CompilerParams(needs_layout_passes=False),
        scratch_types=[
            pltpu.VMEM((K,), jnp.int32),
            pltpu.VMEM((K,), jnp.int32),
            pltpu.VMEM((K,), _f32),
            pltpu.VMEM((K, D), _f32),
            pltpu.VMEM_SHARED((NP, D), _f32),
            pltpu.SemaphoreType.DMA,
        ],
    )(val, row, col, h, znd)


@jax.jit
def kernel(x, edge_index, ep_w1, ep_b1, ep_w2, ep_b2, cheb_w, cheb_b):
    row = edge_index[0]
    col = edge_index[1]
    w2 = ep_w2.reshape(H)
    b2x16 = jnp.broadcast_to(ep_b2.reshape(()), (16,)).astype(_f32)
    zn = jnp.zeros((NP,), _f32)
    znd = jnp.zeros((NP, D), _f32)

    p, q = _compute_pq(x, ep_w1, ep_b1)
    edge_weights, degp = _sc_edge(p, q, row, col, w2, b2x16, zn)
    val, txp = _sc_spmv1(degp, edge_weights, row, col, x, znd)
    tx1, out01 = _tc_mid(x, txp[0, :N], txp[1, :N], cheb_w[0], cheb_w[1])
    txp2 = _sc_spmv2(val, row, col, tx1, znd)
    out = _tc_final(x, out01, txp2[0, :N], txp2[1, :N], cheb_w[2], cheb_b)
    return (out, edge_weights)


# speedup vs baseline: 5.3247x; 1.2138x over previous
import functools

import jax
import jax.numpy as jnp
from jax import lax
from jax.experimental import pallas as pl
from jax.experimental.pallas import tpu as pltpu
from jax.experimental.pallas import tpu_sc as plsc

N = 10000
E = 320000
D = 128
H = 128

NC = 2
NS = 16
NW = NC * NS
NP = 10240
ROWS_PT = NP // NS
EW = E // NW
K = 80
NCHUNK = EW // K

ROW_BLK = 1000

_mesh = plsc.VectorSubcoreMesh(core_axis_name="c", subcore_axis_name="s")
_f32 = jnp.float32


def _pq_body(x_ref, w1a_ref, w1b_ref, b1_ref, p_ref, q_ref):
    x = x_ref[...]
    p_ref[...] = jnp.dot(x, w1a_ref[...].T, preferred_element_type=_f32)
    q_ref[...] = (
        jnp.dot(x, w1b_ref[...].T, preferred_element_type=_f32) + b1_ref[...]
    )


def _compute_pq(x, ep_w1, ep_b1):
    return pl.pallas_call(
        _pq_body,
        grid=(N // ROW_BLK,),
        in_specs=[
            pl.BlockSpec((ROW_BLK, D), lambda i: (i, 0)),
            pl.BlockSpec((H, D), lambda i: (0, 0)),
            pl.BlockSpec((H, D), lambda i: (0, 0)),
            pl.BlockSpec((H,), lambda i: (0,)),
        ],
        out_specs=[
            pl.BlockSpec((ROW_BLK, H), lambda i: (i, 0)),
            pl.BlockSpec((ROW_BLK, H), lambda i: (i, 0)),
        ],
        out_shape=[
            jax.ShapeDtypeStruct((N, H), _f32),
            jax.ShapeDtypeStruct((N, H), _f32),
        ],
    )(x, ep_w1[:, :D], ep_w1[:, D:], ep_b1)


def _mid_body(x_ref, p0_ref, p1_ref, w0_ref, w1_ref, tx1_ref, o_ref):
    t1 = p0_ref[...] + p1_ref[...]
    tx1_ref[...] = t1
    o_ref[...] = (
        jnp.dot(x_ref[...], w0_ref[...].T, preferred_element_type=_f32)
        + jnp.dot(t1, w1_ref[...].T, preferred_element_type=_f32)
    )


def _tc_mid(x, p0, p1, w0, w1):
    wspec = pl.BlockSpec((D, D), lambda i: (0, 0))
    rspec = pl.BlockSpec((ROW_BLK, D), lambda i: (i, 0))
    return pl.pallas_call(
        _mid_body,
        grid=(N // ROW_BLK,),
        in_specs=[rspec, rspec, rspec, wspec, wspec],
        out_specs=[rspec, rspec],
        out_shape=[
            jax.ShapeDtypeStruct((N, D), _f32),
            jax.ShapeDtypeStruct((N, D), _f32),
        ],
    )(x, p0, p1, w0, w1)


def _final_body(x_ref, o01_ref, q0_ref, q1_ref, w2_ref, b_ref, o_ref):
    t2 = 2.0 * (q0_ref[...] + q1_ref[...]) - x_ref[...]
    o_ref[...] = (
        o01_ref[...]
        + jnp.dot(t2, w2_ref[...].T, preferred_element_type=_f32)
        + b_ref[...]
    )


def _tc_final(x, o01, q0, q1, w2, b):
    wspec = pl.BlockSpec((D, D), lambda i: (0, 0))
    rspec = pl.BlockSpec((ROW_BLK, D), lambda i: (i, 0))
    return pl.pallas_call(
        _final_body,
        grid=(N // ROW_BLK,),
        in_specs=[rspec, rspec, rspec, rspec, wspec,
                  pl.BlockSpec((D,), lambda i: (0,))],
        out_specs=rspec,
        out_shape=jax.ShapeDtypeStruct((N, D), _f32),
    )(x, o01, q0, q1, w2, b)


def _rsqrt16(d):
    i = plsc.bitcast(d, jnp.int32)
    i = 0x5F3759DF - lax.shift_right_arithmetic(i, 1)
    y = plsc.bitcast(i, _f32)
    for _ in range(3):
        y = y * (1.5 - 0.5 * d * y * y)
    return y


def _sc_edge_body(p_hbm, q_hbm, row_hbm, col_hbm, w2_hbm, b2_hbm, zn_hbm,
                  w_hbm, degp_hbm,
                  idxrA, idxcA, pbufA, qbufA,
                  idxrB, idxcB, pbufB, qbufB,
                  wbuf, w2v, b2v, deg_sh,
                  semPA, semQA, semPB, semQB):
    c = lax.axis_index("c")
    s = lax.axis_index("s")
    wid = s * NC + c
    pltpu.sync_copy(w2_hbm, w2v)
    pltpu.sync_copy(b2_hbm, b2v)
    sl_tile = pl.ds(s * ROWS_PT, ROWS_PT)
    pltpu.sync_copy(zn_hbm.at[sl_tile], deg_sh.at[sl_tile])
    plsc.subcore_barrier()

    def stage(i, idxr, idxc, pbuf, qbuf, semP, semQ):
        base = wid * EW + i * K
        pltpu.sync_copy(row_hbm.at[pl.ds(base, K)], idxr)
        pltpu.sync_copy(col_hbm.at[pl.ds(base, K)], idxc)
        pltpu.async_copy(p_hbm.at[idxr], pbuf, semP)
        pltpu.async_copy(q_hbm.at[idxc], qbuf, semQ)

    def drain(idxr, idxc, pbuf, qbuf, semP, semQ):
        pltpu.make_async_copy(p_hbm.at[idxr], pbuf, semP).wait()
        pltpu.make_async_copy(q_hbm.at[idxc], qbuf, semQ).wait()

    def compute(i, idxr, pbuf, qbuf):
        def group(g, carry2):
            gbase = g * 16
            e16 = gbase + jnp.arange(16, dtype=jnp.int32)
            acc = b2v[...]
            for j in range(8):
                w2j = w2v[pl.ds(j * 16, 16)]
                for f in range(16):
                    fi = jnp.full((16,), j * 16 + f, jnp.int32)
                    pv = plsc.load_gather(pbuf, [e16, fi])
                    qv = plsc.load_gather(qbuf, [e16, fi])
                    acc = acc + jnp.maximum(pv + qv, 0.0) * w2j[f]
            wbuf[pl.ds(gbase, 16)] = 1.0 / (1.0 + jnp.exp(-acc))
            return carry2

        lax.fori_loop(0, K // 16, group, 0)
        base = wid * EW + i * K
        pltpu.sync_copy(wbuf, w_hbm.at[pl.ds(base, K)])
        pltpu.sync_copy(wbuf, deg_sh.at[idxr], add=True)

    stage(0, idxrA, idxcA, pbufA, qbufA, semPA, semQA)

    def pair(t, carry):
        i0 = 2 * t
        stage(i0 + 1, idxrB, idxcB, pbufB, qbufB, semPB, semQB)
        drain(idxrA, idxcA, pbufA, qbufA, semPA, semQA)
        compute(i0, idxrA, pbufA, qbufA)
        stage(i0 + 2, idxrA, idxcA, pbufA, qbufA, semPA, semQA)
        drain(idxrB, idxcB, pbufB, qbufB, semPB, semQB)
        compute(i0 + 1, idxrB, pbufB, qbufB)
        return carry

    lax.fori_loop(0, NCHUNK // 2, pair, 0)
    drain(idxrA, idxcA, pbufA, qbufA, semPA, semQA)
    compute(NCHUNK - 1, idxrA, pbufA, qbufA)

    plsc.subcore_barrier()
    pltpu.sync_copy(deg_sh.at[sl_tile], degp_hbm.at[c, sl_tile])


def _sc_edge(p, q, row, col, w2, b2x16, zn):
    return pl.kernel(
        _sc_edge_body,
        out_type=[
            jax.ShapeDtypeStruct((E,), _f32),
            jax.ShapeDtypeStruct((NC, NP), _f32),
        ],
        mesh=_mesh,
        compiler_params=pltpu.CompilerParams(needs_layout_passes=False),
        scratch_types=[
            pltpu.VMEM((K,), jnp.int32),
            pltpu.VMEM((K,), jnp.int32),
            pltpu.VMEM((K, H), _f32),
            pltpu.VMEM((K, H), _f32),
            pltpu.VMEM((K,), jnp.int32),
            pltpu.VMEM((K,), jnp.int32),
            pltpu.VMEM((K, H), _f32),
            pltpu.VMEM((K, H), _f32),
            pltpu.VMEM((K,), _f32),
            pltpu.VMEM((H,), _f32),
            pltpu.VMEM((16,), _f32),
            pltpu.VMEM_SHARED((NP,), _f32),
            pltpu.SemaphoreType.DMA,
            pltpu.SemaphoreType.DMA,
            pltpu.SemaphoreType.DMA,
            pltpu.SemaphoreType.DMA,
        ],
    )(p, q, row, col, w2, b2x16, zn)


def _sc_spmv1_body(degp_hbm, w_hbm, row_hbm, col_hbm, x_hbm, znd_hbm,
                   val_hbm, txp_hbm,
                   disv, degb,
                   idxrA, idxcA, wvA, xbufA,
                   idxrB, idxcB, wvB, xbufB,
                   valv, tx_sh, semA, semB):
    c = lax.axis_index("c")
    s = lax.axis_index("s")
    wid = s * NC + c
    sl_tile = pl.ds(s * ROWS_PT, ROWS_PT)
    pltpu.sync_copy(znd_hbm.at[sl_tile], tx_sh.at[sl_tile])
    pltpu.sync_copy(degp_hbm.at[0], disv)
    pltpu.sync_copy(degp_hbm.at[1], degb)

    def dis_step(t, carry):
        sl = pl.ds(t * 16, 16)
        d = disv[sl] + degb[sl]
        disv[sl] = jnp.where(d > 0.0, _rsqrt16(d), 0.0)
        return carry

    lax.fori_loop(0, NP // 16, dis_step, 0)
    plsc.subcore_barrier()

    def stage(i, idxr, idxc, wv, xbuf, sem):
        base = wid * EW + i * K
        pltpu.sync_copy(row_hbm.at[pl.ds(base, K)], idxr)
        pltpu.sync_copy(col_hbm.at[pl.ds(base, K)], idxc)
        pltpu.sync_copy(w_hbm.at[pl.ds(base, K)], wv)
        pltpu.async_copy(x_hbm.at[idxr], xbuf, sem)

    def compute(i, idxr, idxc, wv, xbuf, sem):
        pltpu.make_async_copy(x_hbm.at[idxr], xbuf, sem).wait()
        for g in range(K // 16):
            sl = pl.ds(g * 16, 16)
            r16 = idxr[sl]
            c16 = idxc[sl]
            dr = plsc.load_gather(disv, [r16])
            dc = plsc.load_gather(disv, [c16])
            v16 = -(dr * wv[sl] * dc)
            valv[sl] = jnp.where(r16 == c16, v16 - 1.0, v16)
        base = wid * EW + i * K
        pltpu.sync_copy(valv, val_hbm.at[pl.ds(base, K)])

        @plsc.parallel_loop(0, K // 16, 1, unroll=1)
        def scale(g):
            v16 = valv[pl.ds(g * 16, 16)]
            for l in range(16):
                sv = v16[l]
                e = g * 16 + l
                for j in range(8):
                    sl = pl.ds(j * 16, 16)
                    xbuf[e, sl] = xbuf[e, sl] * sv

        pltpu.sync_copy(xbuf, tx_sh.at[idxc], add=True)

    stage(0, idxrA, idxcA, wvA, xbufA, semA)

    def pair(t, carry):
        i0 = 2 * t
        stage(i0 + 1, idxrB, idxcB, wvB, xbufB, semB)
        compute(i0, idxrA, idxcA, wvA, xbufA, semA)
        stage(i0 + 2, idxrA, idxcA, wvA, xbufA, semA)
        compute(i0 + 1, idxrB, idxcB, wvB, xbufB, semB)
        return carry

    lax.fori_loop(0, NCHUNK // 2, pair, 0)
    compute(NCHUNK - 1, idxrA, idxcA, wvA, xbufA, semA)

    plsc.subcore_barrier()
    pltpu.sync_copy(tx_sh.at[sl_tile], txp_hbm.at[c, sl_tile])


def _sc_spmv1(degp, w, row, col, x, znd):
    return pl.kernel(
        _sc_spmv1_body,
        out_type=[
            jax.ShapeDtypeStruct((E,), _f32),
            jax.ShapeDtypeStruct((NC, NP, D), _f32),
        ],
        mesh=_mesh,
        compiler_params=pltpu.CompilerParams(needs_layout_passes=False),
        scratch_types=[
            pltpu.VMEM((NP,), _f32),
            pltpu.VMEM((NP,), _f32),
            pltpu.VMEM((K,), jnp.int32),
            pltpu.VMEM((K,), jnp.int32),
            pltpu.VMEM((K,), _f32),
            pltpu.VMEM((K, D), _f32),
            pltpu.VMEM((K,), jnp.int32),
            pltpu.VMEM((K,), jnp.int32),
            pltpu.VMEM((K,), _f32),
            pltpu.VMEM((K, D), _f32),
            pltpu.VMEM((K,), _f32),
            pltpu.VMEM_SHARED((NP, D), _f32),
            pltpu.SemaphoreType.DMA,
            pltpu.SemaphoreType.DMA,
        ],
    )(degp, w, row, col, x, znd)


def _sc_spmv2_body(val_hbm, row_hbm, col_hbm, h_hbm, znd_hbm,
                   txp_hbm,
                   idxrA, idxcA, valvA, xbufA,
                   idxrB, idxcB, valvB, xbufB,
                   tx_sh, semA, semB):
    c = lax.axis_index("c")
    s = lax.axis_index("s")
    wid = s * NC + c
    sl_tile = pl.ds(s * ROWS_PT, ROWS_PT)
    pltpu.sync_copy(znd_hbm.at[sl_tile], tx_sh.at[sl_tile])
    plsc.subcore_barrier()

    def stage(i, idxr, idxc, valv, xbuf, sem):
        base = wid * EW + i * K
        pltpu.sync_copy(row_hbm.at[pl.ds(base, K)], idxr)
        pltpu.sync_copy(col_hbm.at[pl.ds(base, K)], idxc)
        pltpu.sync_copy(val_hbm.at[pl.ds(base, K)], valv)
        pltpu.async_copy(h_hbm.at[idxr], xbuf, sem)

    def compute(idxr, idxc, valv, xbuf, sem):
        pltpu.make_async_copy(h_hbm.at[idxr], xbuf, sem).wait()

        @plsc.parallel_loop(0, K // 16, 1, unroll=1)
        def scale(g):
            v16 = valv[pl.ds(g * 16, 16)]
            for l in range(16):
                sv = v16[l]
                e = g * 16 + l
                for j in range(8):
                    sl = pl.ds(j * 16, 16)
                    xbuf[e, sl] = xbuf[e, sl] * sv

        pltpu.sync_copy(xbuf, tx_sh.at[idxc], add=True)

    stage(0, idxrA, idxcA, valvA, xbufA, semA)

    def pair(t, carry):
        i0 = 2 * t
        stage(i0 + 1, idxrB, idxcB, valvB, xbufB, semB)
        compute(idxrA, idxcA, valvA, xbufA, semA)
        stage(i0 + 2, idxrA, idxcA, valvA, xbufA, semA)
        compute(idxrB, idxcB, valvB, xbufB, semB)
        return carry

    lax.fori_loop(0, NCHUNK // 2, pair, 0)
    compute(idxrA, idxcA, valvA, xbufA, semA)

    plsc.subcore_barrier()
    pltpu.sync_copy(tx_sh.at[sl_tile], txp_hbm.at[c, sl_tile])


def _sc_spmv2(val, row, col, h, znd):
    return pl.kernel(
        _sc_spmv2_body,
        out_type=jax.ShapeDtypeStruct((NC, NP, D), _f32),
        mesh=_mesh,
        compiler_params=pltpu.CompilerParams(needs_layout_passes=False),
        scratch_types=[
            pltpu.VMEM((K,), jnp.int32),
            pltpu.VMEM((K,), jnp.int32),
            pltpu.VMEM((K,), _f32),
            pltpu.VMEM((K, D), _f32),
            pltpu.VMEM((K,), jnp.int32),
            pltpu.VMEM((K,), jnp.int32),
            pltpu.VMEM((K,), _f32),
            pltpu.VMEM((K, D), _f32),
            pltpu.VMEM_SHARED((NP, D), _f32),
            pltpu.SemaphoreType.DMA,
            pltpu.SemaphoreType.DMA,
        ],
    )(val, row, col, h, znd)


@jax.jit
def kernel(x, edge_index, ep_w1, ep_b1, ep_w2, ep_b2, cheb_w, cheb_b):
    row = edge_index[0]
    col = edge_index[1]
    w2 = ep_w2.reshape(H)
    b2x16 = jnp.broadcast_to(ep_b2.reshape(()), (16,)).astype(_f32)
    zn = jnp.zeros((NP,), _f32)
    znd = jnp.zeros((NP, D), _f32)

    p, q = _compute_pq(x, ep_w1, ep_b1)
    edge_weights, degp = _sc_edge(p, q, row, col, w2, b2x16, zn)
    val, txp = _sc_spmv1(degp, edge_weights, row, col, x, znd)
    tx1, out01 = _tc_mid(x, txp[0, :N], txp[1, :N], cheb_w[0], cheb_w[1])
    txp2 = _sc_spmv2(val, row, col, tx1, znd)
    out = _tc_final(x, out01, txp2[0, :N], txp2[1, :N], cheb_w[2], cheb_b)
    return (out, edge_weights)


# speedup vs baseline: 5.6543x; 1.0619x over previous
import functools

import jax
import jax.numpy as jnp
from jax import lax
from jax.experimental import pallas as pl
from jax.experimental.pallas import tpu as pltpu
from jax.experimental.pallas import tpu_sc as plsc

N = 10000
E = 320000
D = 128
H = 128

NC = 2
NS = 16
NW = NC * NS
NP = 10240
ROWS_PT = NP // NS
EW = E // NW
K = 80
NCHUNK = EW // K
NQUAD = (NCHUNK - 1) // 4
DEG_WIN = 8

ROW_BLK = 1000

_mesh = plsc.VectorSubcoreMesh(core_axis_name="c", subcore_axis_name="s")
_f32 = jnp.float32
_i32 = jnp.int32


def _pq_body(x_ref, w1a_ref, w1b_ref, b1_ref, p_ref, q_ref):
    x = x_ref[...]
    p_ref[...] = jnp.dot(x, w1a_ref[...].T, preferred_element_type=_f32)
    q_ref[...] = (
        jnp.dot(x, w1b_ref[...].T, preferred_element_type=_f32) + b1_ref[...]
    )


def _compute_pq(x, ep_w1, ep_b1):
    return pl.pallas_call(
        _pq_body,
        grid=(N // ROW_BLK,),
        in_specs=[
            pl.BlockSpec((ROW_BLK, D), lambda i: (i, 0)),
            pl.BlockSpec((H, D), lambda i: (0, 0)),
            pl.BlockSpec((H, D), lambda i: (0, 0)),
            pl.BlockSpec((H,), lambda i: (0,)),
        ],
        out_specs=[
            pl.BlockSpec((ROW_BLK, H), lambda i: (i, 0)),
            pl.BlockSpec((ROW_BLK, H), lambda i: (i, 0)),
        ],
        out_shape=[
            jax.ShapeDtypeStruct((N, H), _f32),
            jax.ShapeDtypeStruct((N, H), _f32),
        ],
    )(x, ep_w1[:, :D], ep_w1[:, D:], ep_b1)


def _mid_body(x_ref, p0_ref, p1_ref, w0_ref, w1_ref, tx1_ref, o_ref):
    t1 = p0_ref[...] + p1_ref[...]
    tx1_ref[...] = t1
    o_ref[...] = (
        jnp.dot(x_ref[...], w0_ref[...].T, preferred_element_type=_f32)
        + jnp.dot(t1, w1_ref[...].T, preferred_element_type=_f32)
    )


def _tc_mid(x, p0, p1, w0, w1):
    wspec = pl.BlockSpec((D, D), lambda i: (0, 0))
    rspec = pl.BlockSpec((ROW_BLK, D), lambda i: (i, 0))
    return pl.pallas_call(
        _mid_body,
        grid=(N // ROW_BLK,),
        in_specs=[rspec, rspec, rspec, wspec, wspec],
        out_specs=[rspec, rspec],
        out_shape=[
            jax.ShapeDtypeStruct((N, D), _f32),
            jax.ShapeDtypeStruct((N, D), _f32),
        ],
    )(x, p0, p1, w0, w1)


def _final_body(x_ref, o01_ref, q0_ref, q1_ref, w2_ref, b_ref, o_ref):
    t2 = 2.0 * (q0_ref[...] + q1_ref[...]) - x_ref[...]
    o_ref[...] = (
        o01_ref[...]
        + jnp.dot(t2, w2_ref[...].T, preferred_element_type=_f32)
        + b_ref[...]
    )


def _tc_final(x, o01, q0, q1, w2, b):
    wspec = pl.BlockSpec((D, D), lambda i: (0, 0))
    rspec = pl.BlockSpec((ROW_BLK, D), lambda i: (i, 0))
    return pl.pallas_call(
        _final_body,
        grid=(N // ROW_BLK,),
        in_specs=[rspec, rspec, rspec, rspec, wspec,
                  pl.BlockSpec((D,), lambda i: (0,))],
        out_specs=rspec,
        out_shape=jax.ShapeDtypeStruct((N, D), _f32),
    )(x, o01, q0, q1, w2, b)


def _rsqrt16(d):
    i = plsc.bitcast(d, _i32)
    i = 0x5F3759DF - lax.shift_right_arithmetic(i, 1)
    y = plsc.bitcast(i, _f32)
    for _ in range(3):
        y = y * (1.5 - 0.5 * d * y * y)
    return y


def _sc_edge_body(p_hbm, q_hbm, row3_hbm, col3_hbm, w2_hbm, b2_hbm, zn_hbm,
                  w_hbm, degp_hbm,
                  rowb, colb, wbuf, w2v, b2v, pbufs, qbufs, deg_sh,
                  semP, semQ, semD):
    c = lax.axis_index("c")
    s = lax.axis_index("s")
    wid = s * NC + c
    pltpu.sync_copy(w2_hbm, w2v)
    pltpu.sync_copy(b2_hbm, b2v)
    sl_tile = pl.ds(s * ROWS_PT, ROWS_PT)
    pltpu.sync_copy(zn_hbm.at[sl_tile], deg_sh.at[sl_tile])
    pltpu.sync_copy(row3_hbm.at[wid], rowb)
    pltpu.sync_copy(col3_hbm.at[wid], colb)
    plsc.subcore_barrier()

    def fire_gather(i, slot):
        pltpu.async_copy(p_hbm.at[rowb.at[i]], pbufs[slot], semP[slot])
        pltpu.async_copy(q_hbm.at[colb.at[i]], qbufs[slot], semQ[slot])

    def wait_gather(i, slot):
        pltpu.make_async_copy(p_hbm.at[rowb.at[i]], pbufs[slot],
                              semP[slot]).wait()
        pltpu.make_async_copy(q_hbm.at[colb.at[i]], qbufs[slot],
                              semQ[slot]).wait()

    def compute(i, slot):
        pbuf = pbufs[slot]
        qbuf = qbufs[slot]

        def group(g, carry2):
            gbase = g * 16
            e16 = gbase + jnp.arange(16, dtype=_i32)
            acc = b2v[...]
            for j in range(8):
                w2j = w2v[pl.ds(j * 16, 16)]
                for f in range(16):
                    fi = jnp.full((16,), j * 16 + f, _i32)
                    pv = plsc.load_gather(pbuf, [e16, fi])
                    qv = plsc.load_gather(qbuf, [e16, fi])
                    acc = acc + jnp.maximum(pv + qv, 0.0) * w2j[f]
            wbuf[pl.ds(i * K + gbase, 16)] = 1.0 / (1.0 + jnp.exp(-acc))
            return carry2

        lax.fori_loop(0, K // 16, group, 0)
        return pltpu.async_copy(wbuf.at[pl.ds(i * K, K)],
                                deg_sh.at[rowb.at[i]], semD, add=True)

    fire_gather(0, 0)
    fire_gather(1, 1)

    def quad(t, carry):
        descs = []
        for k in range(4):
            i = 4 * t + k
            slot = k
            nslot = (k + 2) % 4
            wait_gather(i, slot)

            @pl.when(i + 2 < NCHUNK)
            def _():
                fire_gather(i + 2, nslot)

            descs.append(compute(i, slot))
        for d in descs:
            d.wait()
        return carry

    lax.fori_loop(0, NQUAD, quad, 0)
    i_tail = NCHUNK - 1
    wait_gather(i_tail, i_tail % 4)
    compute(i_tail, i_tail % 4).wait()
    pltpu.sync_copy(wbuf, w_hbm.at[pl.ds(wid * EW, EW)])
    plsc.subcore_barrier()
    pltpu.sync_copy(deg_sh.at[sl_tile], degp_hbm.at[c, sl_tile])


def _sc_edge(p, q, row3, col3, w2, b2x16, zn):
    return pl.kernel(
        _sc_edge_body,
        out_type=[
            jax.ShapeDtypeStruct((E,), _f32),
            jax.ShapeDtypeStruct((NC, NP), _f32),
        ],
        mesh=_mesh,
        compiler_params=pltpu.CompilerParams(needs_layout_passes=False),
        scratch_types=[
            pltpu.VMEM((NCHUNK, K), _i32),
            pltpu.VMEM((NCHUNK, K), _i32),
            pltpu.VMEM((EW,), _f32),
            pltpu.VMEM((H,), _f32),
            pltpu.VMEM((16,), _f32),
            [pltpu.VMEM((K, H), _f32)] * 4,
            [pltpu.VMEM((K, H), _f32)] * 4,
            pltpu.VMEM_SHARED((NP,), _f32),
            [pltpu.SemaphoreType.DMA] * 4,
            [pltpu.SemaphoreType.DMA] * 4,
            pltpu.SemaphoreType.DMA,
        ],
    )(p, q, row3, col3, w2, b2x16, zn)


def _scale_chunk(valv, xbuf):
    @plsc.parallel_loop(0, K // 16, 1, unroll=1)
    def scale(g):
        v16 = valv[pl.ds(g * 16, 16)]
        for l in range(16):
            sv = v16[l]
            e = g * 16 + l
            for j in range(8):
                sl = pl.ds(j * 16, 16)
                xbuf[e, sl] = xbuf[e, sl] * sv


def _sc_spmv1_body(degp_hbm, w_hbm, row_hbm, col_hbm, x_hbm, znd_hbm,
                   val_hbm, txp_hbm,
                   disv, degb,
                   idxrA, idxcA, wvA, xbufA,
                   idxrB, idxcB, wvB, xbufB,
                   valv, tx_sh, semA, semB):
    c = lax.axis_index("c")
    s = lax.axis_index("s")
    wid = s * NC + c
    sl_tile = pl.ds(s * ROWS_PT, ROWS_PT)
    pltpu.sync_copy(znd_hbm.at[sl_tile], tx_sh.at[sl_tile])
    pltpu.sync_copy(degp_hbm.at[0], disv)
    pltpu.sync_copy(degp_hbm.at[1], degb)

    def dis_step(t, carry):
        sl = pl.ds(t * 16, 16)
        d = disv[sl] + degb[sl]
        disv[sl] = jnp.where(d > 0.0, _rsqrt16(d), 0.0)
        return carry

    lax.fori_loop(0, NP // 16, dis_step, 0)
    plsc.subcore_barrier()

    def stage(i, idxr, idxc, wv, xbuf, sem):
        base = wid * EW + i * K
        pltpu.sync_copy(row_hbm.at[pl.ds(base, K)], idxr)
        pltpu.sync_copy(col_hbm.at[pl.ds(base, K)], idxc)
        pltpu.sync_copy(w_hbm.at[pl.ds(base, K)], wv)
        pltpu.async_copy(x_hbm.at[idxr], xbuf, sem)

    def compute(i, idxr, idxc, wv, xbuf, sem):
        pltpu.make_async_copy(x_hbm.at[idxr], xbuf, sem).wait()
        for g in range(K // 16):
            sl = pl.ds(g * 16, 16)
            r16 = idxr[sl]
            c16 = idxc[sl]
            dr = plsc.load_gather(disv, [r16])
            dc = plsc.load_gather(disv, [c16])
            v16 = -(dr * wv[sl] * dc)
            valv[sl] = jnp.where(r16 == c16, v16 - 1.0, v16)
        base = wid * EW + i * K
        pltpu.sync_copy(valv, val_hbm.at[pl.ds(base, K)])
        _scale_chunk(valv, xbuf)
        pltpu.sync_copy(xbuf, tx_sh.at[idxc], add=True)

    stage(0, idxrA, idxcA, wvA, xbufA, semA)

    def pair(t, carry):
        i0 = 2 * t
        stage(i0 + 1, idxrB, idxcB, wvB, xbufB, semB)
        compute(i0, idxrA, idxcA, wvA, xbufA, semA)
        stage(i0 + 2, idxrA, idxcA, wvA, xbufA, semA)
        compute(i0 + 1, idxrB, idxcB, wvB, xbufB, semB)
        return carry

    lax.fori_loop(0, NCHUNK // 2, pair, 0)
    compute(NCHUNK - 1, idxrA, idxcA, wvA, xbufA, semA)

    plsc.subcore_barrier()
    pltpu.sync_copy(tx_sh.at[sl_tile], txp_hbm.at[c, sl_tile])


def _sc_spmv1(degp, w, row, col, x, znd):
    return pl.kernel(
        _sc_spmv1_body,
        out_type=[
            jax.ShapeDtypeStruct((E,), _f32),
            jax.ShapeDtypeStruct((NC, NP, D), _f32),
        ],
        mesh=_mesh,
        compiler_params=pltpu.CompilerParams(needs_layout_passes=False),
        scratch_types=[
            pltpu.VMEM((NP,), _f32),
            pltpu.VMEM((NP,), _f32),
            pltpu.VMEM((K,), jnp.int32),
            pltpu.VMEM((K,), jnp.int32),
            pltpu.VMEM((K,), _f32),
            pltpu.VMEM((K, D), _f32),
            pltpu.VMEM((K,), jnp.int32),
            pltpu.VMEM((K,), jnp.int32),
            pltpu.VMEM((K,), _f32),
            pltpu.VMEM((K, D), _f32),
            pltpu.VMEM((K,), _f32),
            pltpu.VMEM_SHARED((NP, D), _f32),
            pltpu.SemaphoreType.DMA,
            pltpu.SemaphoreType.DMA,
        ],
    )(degp, w, row, col, x, znd)


def _sc_spmv2_body(val_hbm, row_hbm, col_hbm, h_hbm, znd_hbm,
                   txp_hbm,
                   idxrA, idxcA, valvA, xbufA,
                   idxrB, idxcB, valvB, xbufB,
                   tx_sh, semA, semB):
    c = lax.axis_index("c")
    s = lax.axis_index("s")
    wid = s * NC + c
    sl_tile = pl.ds(s * ROWS_PT, ROWS_PT)
    pltpu.sync_copy(znd_hbm.at[sl_tile], tx_sh.at[sl_tile])
    plsc.subcore_barrier()

    def stage(i, idxr, idxc, valv, xbuf, sem):
        base = wid * EW + i * K
        pltpu.sync_copy(row_hbm.at[pl.ds(base, K)], idxr)
        pltpu.sync_copy(col_hbm.at[pl.ds(base, K)], idxc)
        pltpu.sync_copy(val_hbm.at[pl.ds(base, K)], valv)
        pltpu.async_copy(h_hbm.at[idxr], xbuf, sem)

    def compute(idxr, idxc, valv, xbuf, sem):
        pltpu.make_async_copy(h_hbm.at[idxr], xbuf, sem).wait()
        _scale_chunk(valv, xbuf)
        pltpu.sync_copy(xbuf, tx_sh.at[idxc], add=True)

    stage(0, idxrA, idxcA, valvA, xbufA, semA)

    def pair(t, carry):
        i0 = 2 * t
        stage(i0 + 1, idxrB, idxcB, valvB, xbufB, semB)
        compute(idxrA, idxcA, valvA, xbufA, semA)
        stage(i0 + 2, idxrA, idxcA, valvA, xbufA, semA)
        compute(idxrB, idxcB, valvB, xbufB, semB)
        return carry

    lax.fori_loop(0, NCHUNK // 2, pair, 0)
    compute(idxrA, idxcA, valvA, xbufA, semA)

    plsc.subcore_barrier()
    pltpu.sync_copy(tx_sh.at[sl_tile], txp_hbm.at[c, sl_tile])


def _sc_spmv2(val, row, col, h, znd):
    return pl.kernel(
        _sc_spmv2_body,
        out_type=jax.ShapeDtypeStruct((NC, NP, D), _f32),
        mesh=_mesh,
        compiler_params=pltpu.CompilerParams(needs_layout_passes=False),
        scratch_types=[
            pltpu.VMEM((K,), jnp.int32),
            pltpu.VMEM((K,), jnp.int32),
            pltpu.VMEM((K,), _f32),
            pltpu.VMEM((K, D), _f32),
            pltpu.VMEM((K,), jnp.int32),
            pltpu.VMEM((K,), jnp.int32),
            pltpu.VMEM((K,), _f32),
            pltpu.VMEM((K, D), _f32),
            pltpu.VMEM_SHARED((NP, D), _f32),
            pltpu.SemaphoreType.DMA,
            pltpu.SemaphoreType.DMA,
        ],
    )(val, row, col, h, znd)


def _dis_body(d0_ref, d1_ref, o_ref):
    d = d0_ref[...] + d1_ref[...]
    o_ref[...] = jnp.where(d > 0.0, lax.rsqrt(d), 0.0)


def _tc_dis(degp):
    d2 = degp.reshape(NC, NP // 128, 128)
    out = pl.pallas_call(
        _dis_body,
        grid=(1,),
        in_specs=[
            pl.BlockSpec((NP // 128, 128), lambda i: (0, 0)),
            pl.BlockSpec((NP // 128, 128), lambda i: (0, 0)),
        ],
        out_specs=pl.BlockSpec((NP // 128, 128), lambda i: (0, 0)),
        out_shape=jax.ShapeDtypeStruct((NP // 128, 128), _f32),
    )(d2[0], d2[1])
    return out.reshape(NP)


@jax.jit
def kernel(x, edge_index, ep_w1, ep_b1, ep_w2, ep_b2, cheb_w, cheb_b):
    row3 = edge_index[0].reshape(NW, NCHUNK, K)
    col3 = edge_index[1].reshape(NW, NCHUNK, K)
    w2 = ep_w2.reshape(H)
    b2x16 = jnp.broadcast_to(ep_b2.reshape(()), (16,)).astype(_f32)
    zn = jnp.zeros((NP,), _f32)
    znd = jnp.zeros((NP, D), _f32)

    p, q = _compute_pq(x, ep_w1, ep_b1)
    row = edge_index[0]
    col = edge_index[1]
    edge_weights, degp = _sc_edge(p, q, row3, col3, w2, b2x16, zn)
    val, txp = _sc_spmv1(degp, edge_weights, row, col, x, znd)
    tx1, out01 = _tc_mid(x, txp[0, :N], txp[1, :N], cheb_w[0], cheb_w[1])
    txp2 = _sc_spmv2(val, row, col, tx1, znd)
    out = _tc_final(x, out01, txp2[0, :N], txp2[1, :N], cheb_w[2], cheb_b)
    return (out, edge_weights)


# speedup vs baseline: 12.2736x; 2.1707x over previous
import functools

import jax
import jax.numpy as jnp
from jax import lax
from jax.experimental import pallas as pl
from jax.experimental.pallas import tpu as pltpu
from jax.experimental.pallas import tpu_sc as plsc

N = 10000
E = 320000
D = 128
H = 128

NC = 2
NS = 16
NW = NC * NS
NP = 10240
ROWS_PT = NP // NS
EW = E // NW
K = 80
NCHUNK = EW // K
NQUAD = (NCHUNK - 1) // 4
DEG_WIN = 8

ROW_BLK = 1000

_mesh = plsc.VectorSubcoreMesh(core_axis_name="c", subcore_axis_name="s")
_f32 = jnp.float32
_i32 = jnp.int32


def _pq_body(x_ref, w1a_ref, w1b_ref, b1_ref, p_ref, q_ref):
    x = x_ref[...]
    p_ref[...] = jnp.dot(x, w1a_ref[...].T, preferred_element_type=_f32)
    q_ref[...] = (
        jnp.dot(x, w1b_ref[...].T, preferred_element_type=_f32) + b1_ref[...]
    )


def _compute_pq(x, ep_w1, ep_b1):
    return pl.pallas_call(
        _pq_body,
        grid=(N // ROW_BLK,),
        in_specs=[
            pl.BlockSpec((ROW_BLK, D), lambda i: (i, 0)),
            pl.BlockSpec((H, D), lambda i: (0, 0)),
            pl.BlockSpec((H, D), lambda i: (0, 0)),
            pl.BlockSpec((H,), lambda i: (0,)),
        ],
        out_specs=[
            pl.BlockSpec((ROW_BLK, H), lambda i: (i, 0)),
            pl.BlockSpec((ROW_BLK, H), lambda i: (i, 0)),
        ],
        out_shape=[
            jax.ShapeDtypeStruct((N, H), _f32),
            jax.ShapeDtypeStruct((N, H), _f32),
        ],
    )(x, ep_w1[:, :D], ep_w1[:, D:], ep_b1)


def _mid_body(x_ref, p0_ref, p1_ref, w0_ref, w1_ref, tx1_ref, o_ref):
    t1 = p0_ref[...] + p1_ref[...]
    tx1_ref[...] = t1
    o_ref[...] = (
        jnp.dot(x_ref[...], w0_ref[...].T, preferred_element_type=_f32)
        + jnp.dot(t1, w1_ref[...].T, preferred_element_type=_f32)
    )


def _tc_mid(x, p0, p1, w0, w1):
    wspec = pl.BlockSpec((D, D), lambda i: (0, 0))
    rspec = pl.BlockSpec((ROW_BLK, D), lambda i: (i, 0))
    return pl.pallas_call(
        _mid_body,
        grid=(N // ROW_BLK,),
        in_specs=[rspec, rspec, rspec, wspec, wspec],
        out_specs=[rspec, rspec],
        out_shape=[
            jax.ShapeDtypeStruct((N, D), _f32),
            jax.ShapeDtypeStruct((N, D), _f32),
        ],
    )(x, p0, p1, w0, w1)


def _final_body(x_ref, o01_ref, q0_ref, q1_ref, w2_ref, b_ref, o_ref):
    t2 = 2.0 * (q0_ref[...] + q1_ref[...]) - x_ref[...]
    o_ref[...] = (
        o01_ref[...]
        + jnp.dot(t2, w2_ref[...].T, preferred_element_type=_f32)
        + b_ref[...]
    )


def _tc_final(x, o01, q0, q1, w2, b):
    wspec = pl.BlockSpec((D, D), lambda i: (0, 0))
    rspec = pl.BlockSpec((ROW_BLK, D), lambda i: (i, 0))
    return pl.pallas_call(
        _final_body,
        grid=(N // ROW_BLK,),
        in_specs=[rspec, rspec, rspec, rspec, wspec,
                  pl.BlockSpec((D,), lambda i: (0,))],
        out_specs=rspec,
        out_shape=jax.ShapeDtypeStruct((N, D), _f32),
    )(x, o01, q0, q1, w2, b)


def _rsqrt16(d):
    i = plsc.bitcast(d, _i32)
    i = 0x5F3759DF - lax.shift_right_arithmetic(i, 1)
    y = plsc.bitcast(i, _f32)
    for _ in range(3):
        y = y * (1.5 - 0.5 * d * y * y)
    return y


def _sc_edge_body(p_hbm, q_hbm, row3_hbm, col3_hbm, w2_hbm, b2_hbm, zn_hbm,
                  w_hbm, degp_hbm,
                  rowb, colb, wbuf, w2v, b2v, pbufs, qbufs, deg_sh,
                  semP, semQ, semD):
    c = lax.axis_index("c")
    s = lax.axis_index("s")
    wid = s * NC + c
    pltpu.sync_copy(w2_hbm, w2v.at[pl.ds(0, H)])
    pltpu.sync_copy(w2_hbm.at[pl.ds(0, 16)], w2v.at[pl.ds(H, 16)])
    pltpu.sync_copy(b2_hbm, b2v)
    sl_tile = pl.ds(s * ROWS_PT, ROWS_PT)
    pltpu.sync_copy(zn_hbm.at[sl_tile], deg_sh.at[sl_tile])
    pltpu.sync_copy(row3_hbm.at[wid], rowb)
    pltpu.sync_copy(col3_hbm.at[wid], colb)
    plsc.subcore_barrier()

    def fire_gather(i, slot):
        pltpu.async_copy(p_hbm.at[rowb.at[i]], pbufs[slot], semP[slot])
        pltpu.async_copy(q_hbm.at[colb.at[i]], qbufs[slot], semQ[slot])

    def wait_gather(i, slot):
        pltpu.make_async_copy(p_hbm.at[rowb.at[i]], pbufs[slot],
                              semP[slot]).wait()
        pltpu.make_async_copy(q_hbm.at[colb.at[i]], qbufs[slot],
                              semQ[slot]).wait()

    def compute(i, slot):
        pbuf = pbufs[slot]
        qbuf = qbufs[slot]

        def group(g, carry2):
            gbase = g * 16
            e16 = gbase + jnp.arange(16, dtype=_i32)
            lanes = jnp.arange(16, dtype=_i32)

            def fblock(jf, acc):
                fb = jf * 16
                for f2 in range(16):
                    col16 = (lanes + (fb + f2)) & 127
                    pv = plsc.load_gather(pbuf, [e16, col16])
                    qv = plsc.load_gather(qbuf, [e16, col16])
                    acc = acc + (jnp.maximum(pv + qv, 0.0)
                                 * w2v[pl.ds(fb + f2, 16)])
                return acc

            acc = lax.fori_loop(0, H // 16, fblock, b2v[...])
            wbuf[pl.ds(i * K + gbase, 16)] = 1.0 / (1.0 + jnp.exp(-acc))
            return carry2

        lax.fori_loop(0, K // 16, group, 0)
        return pltpu.async_copy(wbuf.at[pl.ds(i * K, K)],
                                deg_sh.at[rowb.at[i]], semD, add=True)

    fire_gather(0, 0)
    fire_gather(1, 1)

    def quad(t, carry):
        descs = []
        for k in range(4):
            i = 4 * t + k
            slot = k
            nslot = (k + 2) % 4
            wait_gather(i, slot)

            @pl.when(i + 2 < NCHUNK)
            def _():
                fire_gather(i + 2, nslot)

            descs.append(compute(i, slot))
        for d in descs:
            d.wait()
        return carry

    lax.fori_loop(0, NQUAD, quad, 0)
    i_tail = NCHUNK - 1
    wait_gather(i_tail, i_tail % 4)
    compute(i_tail, i_tail % 4).wait()
    pltpu.sync_copy(wbuf, w_hbm.at[pl.ds(wid * EW, EW)])
    plsc.subcore_barrier()
    pltpu.sync_copy(deg_sh.at[sl_tile], degp_hbm.at[c, sl_tile])


def _sc_edge(p, q, row3, col3, w2, b2x16, zn):
    return pl.kernel(
        _sc_edge_body,
        out_type=[
            jax.ShapeDtypeStruct((E,), _f32),
            jax.ShapeDtypeStruct((NC, NP), _f32),
        ],
        mesh=_mesh,
        compiler_params=pltpu.CompilerParams(needs_layout_passes=False),
        scratch_types=[
            pltpu.VMEM((NCHUNK, K), _i32),
            pltpu.VMEM((NCHUNK, K), _i32),
            pltpu.VMEM((EW,), _f32),
            pltpu.VMEM((H + 16,), _f32),
            pltpu.VMEM((16,), _f32),
            [pltpu.VMEM((K, H), _f32)] * 4,
            [pltpu.VMEM((K, H), _f32)] * 4,
            pltpu.VMEM_SHARED((NP,), _f32),
            [pltpu.SemaphoreType.DMA] * 4,
            [pltpu.SemaphoreType.DMA] * 4,
            pltpu.SemaphoreType.DMA,
        ],
    )(p, q, row3, col3, w2, b2x16, zn)


def _scale_chunk(valv, xbuf):
    @plsc.parallel_loop(0, K // 16, 1, unroll=1)
    def scale(g):
        v16 = valv[pl.ds(g * 16, 16)]
        for l in range(16):
            sv = v16[l]
            e = g * 16 + l
            for j in range(8):
                sl = pl.ds(j * 16, 16)
                xbuf[e, sl] = xbuf[e, sl] * sv


def _sc_spmv1_body(degp_hbm, w_hbm, row_hbm, col_hbm, x_hbm, znd_hbm,
                   val_hbm, txp_hbm,
                   disv, degb,
                   idxrA, idxcA, wvA, xbufA,
                   idxrB, idxcB, wvB, xbufB,
                   valv, tx_sh, semA, semB):
    c = lax.axis_index("c")
    s = lax.axis_index("s")
    wid = s * NC + c
    sl_tile = pl.ds(s * ROWS_PT, ROWS_PT)
    pltpu.sync_copy(znd_hbm.at[sl_tile], tx_sh.at[sl_tile])
    pltpu.sync_copy(degp_hbm.at[0], disv)
    pltpu.sync_copy(degp_hbm.at[1], degb)

    def dis_step(t, carry):
        sl = pl.ds(t * 16, 16)
        d = disv[sl] + degb[sl]
        disv[sl] = jnp.where(d > 0.0, _rsqrt16(d), 0.0)
        return carry

    lax.fori_loop(0, NP // 16, dis_step, 0)
    plsc.subcore_barrier()

    def stage(i, idxr, idxc, wv, xbuf, sem):
        base = wid * EW + i * K
        pltpu.sync_copy(row_hbm.at[pl.ds(base, K)], idxr)
        pltpu.sync_copy(col_hbm.at[pl.ds(base, K)], idxc)
        pltpu.sync_copy(w_hbm.at[pl.ds(base, K)], wv)
        pltpu.async_copy(x_hbm.at[idxr], xbuf, sem)

    def compute(i, idxr, idxc, wv, xbuf, sem):
        pltpu.make_async_copy(x_hbm.at[idxr], xbuf, sem).wait()
        for g in range(K // 16):
            sl = pl.ds(g * 16, 16)
            r16 = idxr[sl]
            c16 = idxc[sl]
            dr = plsc.load_gather(disv, [r16])
            dc = plsc.load_gather(disv, [c16])
            v16 = -(dr * wv[sl] * dc)
            valv[sl] = jnp.where(r16 == c16, v16 - 1.0, v16)
        base = wid * EW + i * K
        pltpu.sync_copy(valv, val_hbm.at[pl.ds(base, K)])
        _scale_chunk(valv, xbuf)
        pltpu.sync_copy(xbuf, tx_sh.at[idxc], add=True)

    stage(0, idxrA, idxcA, wvA, xbufA, semA)

    def pair(t, carry):
        i0 = 2 * t
        stage(i0 + 1, idxrB, idxcB, wvB, xbufB, semB)
        compute(i0, idxrA, idxcA, wvA, xbufA, semA)
        stage(i0 + 2, idxrA, idxcA, wvA, xbufA, semA)
        compute(i0 + 1, idxrB, idxcB, wvB, xbufB, semB)
        return carry

    lax.fori_loop(0, NCHUNK // 2, pair, 0)
    compute(NCHUNK - 1, idxrA, idxcA, wvA, xbufA, semA)

    plsc.subcore_barrier()
    pltpu.sync_copy(tx_sh.at[sl_tile], txp_hbm.at[c, sl_tile])


def _sc_spmv1(degp, w, row, col, x, znd):
    return pl.kernel(
        _sc_spmv1_body,
        out_type=[
            jax.ShapeDtypeStruct((E,), _f32),
            jax.ShapeDtypeStruct((NC, NP, D), _f32),
        ],
        mesh=_mesh,
        compiler_params=pltpu.CompilerParams(needs_layout_passes=False),
        scratch_types=[
            pltpu.VMEM((NP,), _f32),
            pltpu.VMEM((NP,), _f32),
            pltpu.VMEM((K,), jnp.int32),
            pltpu.VMEM((K,), jnp.int32),
            pltpu.VMEM((K,), _f32),
            pltpu.VMEM((K, D), _f32),
            pltpu.VMEM((K,), jnp.int32),
            pltpu.VMEM((K,), jnp.int32),
            pltpu.VMEM((K,), _f32),
            pltpu.VMEM((K, D), _f32),
            pltpu.VMEM((K,), _f32),
            pltpu.VMEM_SHARED((NP, D), _f32),
            pltpu.SemaphoreType.DMA,
            pltpu.SemaphoreType.DMA,
        ],
    )(degp, w, row, col, x, znd)


def _sc_spmv2_body(val_hbm, row_hbm, col_hbm, h_hbm, znd_hbm,
                   txp_hbm,
                   idxrA, idxcA, valvA, xbufA,
                   idxrB, idxcB, valvB, xbufB,
                   tx_sh, semA, semB):
    c = lax.axis_index("c")
    s = lax.axis_index("s")
    wid = s * NC + c
    sl_tile = pl.ds(s * ROWS_PT, ROWS_PT)
    pltpu.sync_copy(znd_hbm.at[sl_tile], tx_sh.at[sl_tile])
    plsc.subcore_barrier()

    def stage(i, idxr, idxc, valv, xbuf, sem):
        base = wid * EW + i * K
        pltpu.sync_copy(row_hbm.at[pl.ds(base, K)], idxr)
        pltpu.sync_copy(col_hbm.at[pl.ds(base, K)], idxc)
        pltpu.sync_copy(val_hbm.at[pl.ds(base, K)], valv)
        pltpu.async_copy(h_hbm.at[idxr], xbuf, sem)

    def compute(idxr, idxc, valv, xbuf, sem):
        pltpu.make_async_copy(h_hbm.at[idxr], xbuf, sem).wait()
        _scale_chunk(valv, xbuf)
        pltpu.sync_copy(xbuf, tx_sh.at[idxc], add=True)

    stage(0, idxrA, idxcA, valvA, xbufA, semA)

    def pair(t, carry):
        i0 = 2 * t
        stage(i0 + 1, idxrB, idxcB, valvB, xbufB, semB)
        compute(idxrA, idxcA, valvA, xbufA, semA)
        stage(i0 + 2, idxrA, idxcA, valvA, xbufA, semA)
        compute(idxrB, idxcB, valvB, xbufB, semB)
        return carry

    lax.fori_loop(0, NCHUNK // 2, pair, 0)
    compute(idxrA, idxcA, valvA, xbufA, semA)

    plsc.subcore_barrier()
    pltpu.sync_copy(tx_sh.at[sl_tile], txp_hbm.at[c, sl_tile])


def _sc_spmv2(val, row, col, h, znd):
    return pl.kernel(
        _sc_spmv2_body,
        out_type=jax.ShapeDtypeStruct((NC, NP, D), _f32),
        mesh=_mesh,
        compiler_params=pltpu.CompilerParams(needs_layout_passes=False),
        scratch_types=[
            pltpu.VMEM((K,), jnp.int32),
            pltpu.VMEM((K,), jnp.int32),
            pltpu.VMEM((K,), _f32),
            pltpu.VMEM((K, D), _f32),
            pltpu.VMEM((K,), jnp.int32),
            pltpu.VMEM((K,), jnp.int32),
            pltpu.VMEM((K,), _f32),
            pltpu.VMEM((K, D), _f32),
            pltpu.VMEM_SHARED((NP, D), _f32),
            pltpu.SemaphoreType.DMA,
            pltpu.SemaphoreType.DMA,
        ],
    )(val, row, col, h, znd)


def _dis_body(d0_ref, d1_ref, o_ref):
    d = d0_ref[...] + d1_ref[...]
    o_ref[...] = jnp.where(d > 0.0, lax.rsqrt(d), 0.0)


def _tc_dis(degp):
    d2 = degp.reshape(NC, NP // 128, 128)
    out = pl.pallas_call(
        _dis_body,
        grid=(1,),
        in_specs=[
            pl.BlockSpec((NP // 128, 128), lambda i: (0, 0)),
            pl.BlockSpec((NP // 128, 128), lambda i: (0, 0)),
        ],
        out_specs=pl.BlockSpec((NP // 128, 128), lambda i: (0, 0)),
        out_shape=jax.ShapeDtypeStruct((NP // 128, 128), _f32),
    )(d2[0], d2[1])
    return out.reshape(NP)


@jax.jit
def kernel(x, edge_index, ep_w1, ep_b1, ep_w2, ep_b2, cheb_w, cheb_b):
    row3 = edge_index[0].reshape(NW, NCHUNK, K)
    col3 = edge_index[1].reshape(NW, NCHUNK, K)
    w2 = ep_w2.reshape(H)
    b2x16 = jnp.broadcast_to(ep_b2.reshape(()), (16,)).astype(_f32)
    zn = jnp.zeros((NP,), _f32)
    znd = jnp.zeros((NP, D), _f32)

    p, q = _compute_pq(x, ep_w1, ep_b1)
    row = edge_index[0]
    col = edge_index[1]
    edge_weights, degp = _sc_edge(p, q, row3, col3, w2, b2x16, zn)
    val, txp = _sc_spmv1(degp, edge_weights, row, col, x, znd)
    tx1, out01 = _tc_mid(x, txp[0, :N], txp[1, :N], cheb_w[0], cheb_w[1])
    txp2 = _sc_spmv2(val, row, col, tx1, znd)
    out = _tc_final(x, out01, txp2[0, :N], txp2[1, :N], cheb_w[2], cheb_b)
    return (out, edge_weights)
